# baseline (speedup 1.0000x reference)
import functools

import jax
import jax.numpy as jnp
from jax.experimental import pallas as pl
from jax.experimental.pallas import tpu as pltpu


def _pool_kernel(x_ref, keep_ref, pooled_ref):
    ni = pl.program_id(1)

    @pl.when(ni == 0)
    def _init():
        pooled_ref[...] = jnp.zeros_like(pooled_ref)

    keep = keep_ref[...]
    x = x_ref[...]
    pooled_ref[...] += jnp.sum(x * keep[:, :, None], axis=1)


def _proj_kernel(pooled_ref, keep_ref, hef_ref, w1_ref, b1_ref, w2_ref,
                 out_ref, xl_ref, h_ref, *, tile_d):
    di = pl.program_id(1)

    @pl.when(di == 0)
    def _scalars():
        pooled = pooled_ref[...]
        den = jnp.maximum(jnp.sum(keep_ref[...], axis=1, keepdims=True), 1.0)
        mean = pooled / den
        hef = hef_ref[...]
        sf = jnp.sum(jnp.where(hef != 0.0, 0.0, mean), axis=1, keepdims=True)
        sf = jnp.where(sf == 0.0, 1.0, sf)
        scale = 10000.0 / (den * sf)
        xl_ref[...] = jnp.log1p(pooled * scale)
        h_ref[...] = jnp.zeros_like(h_ref)

    xl = xl_ref[:, pl.ds(di * tile_d, tile_d)]
    h_ref[...] += jnp.dot(xl, w1_ref[...], preferred_element_type=jnp.float32)

    @pl.when(di == pl.num_programs(1) - 1)
    def _finalize():
        h = jnp.maximum(h_ref[...] + b1_ref[...], 0.0)
        out_ref[0] = jnp.dot(h, w2_ref[...],
                             preferred_element_type=jnp.float32)


def kernel(x, x_mask, he_mask, w1, b1, w2, b2):
    B, N, D = x.shape
    M = w1.shape[1]
    f32 = jnp.float32
    x = x.astype(f32)

    keep = (~x_mask).astype(f32)
    hef = he_mask.astype(f32)[None, :]
    b1r = b1.astype(f32)[None, :]
    b2r = b2.astype(f32)[None, :]
    w1 = w1.astype(f32)
    w2 = w2.astype(f32)

    td = min(D, 512)
    tn = min(N, 512)
    nd, nn = D // td, N // tn
    assert nd * td == D and nn * tn == N

    pooled = pl.pallas_call(
        _pool_kernel,
        out_shape=jax.ShapeDtypeStruct((B, D), f32),
        grid_spec=pltpu.PrefetchScalarGridSpec(
            num_scalar_prefetch=0,
            grid=(nd, nn),
            in_specs=[
                pl.BlockSpec((B, tn, td), lambda di, ni: (0, ni, di)),
                pl.BlockSpec((B, tn), lambda di, ni: (0, ni)),
            ],
            out_specs=pl.BlockSpec((B, td), lambda di, ni: (0, di)),
        ),
        compiler_params=pltpu.CompilerParams(
            dimension_semantics=("parallel", "arbitrary"),
            vmem_limit_bytes=48 * 1024 * 1024,
        ),
    )(x, keep)

    mp = 2 if M % 256 == 0 else 1
    mh = M // mp
    td2 = min(D, 512)
    nd2 = D // td2

    y = pl.pallas_call(
        functools.partial(_proj_kernel, tile_d=td2),
        out_shape=jax.ShapeDtypeStruct((mp, B, M), f32),
        grid_spec=pltpu.PrefetchScalarGridSpec(
            num_scalar_prefetch=0,
            grid=(mp, nd2),
            in_specs=[
                pl.BlockSpec((B, D), lambda mc, di: (0, 0)),
                pl.BlockSpec((B, N), lambda mc, di: (0, 0)),
                pl.BlockSpec((1, D), lambda mc, di: (0, 0)),
                pl.BlockSpec((td2, mh), lambda mc, di: (di, mc)),
                pl.BlockSpec((1, mh), lambda mc, di: (0, mc)),
                pl.BlockSpec((mh, M), lambda mc, di: (mc, 0)),
            ],
            out_specs=pl.BlockSpec((1, B, M), lambda mc, di: (mc, 0, 0)),
            scratch_shapes=[
                pltpu.VMEM((B, D), f32),
                pltpu.VMEM((B, mh), f32),
            ],
        ),
        compiler_params=pltpu.CompilerParams(
            dimension_semantics=("parallel", "arbitrary"),
            vmem_limit_bytes=48 * 1024 * 1024,
        ),
    )(pooled, keep, hef, w1, b1r, w2)
    return jnp.sum(y, axis=0) + b2r


# speedup vs baseline: 1.0139x; 1.0139x over previous
import functools

import jax
import jax.numpy as jnp
from jax.experimental import pallas as pl
from jax.experimental.pallas import tpu as pltpu


def _pool_kernel(x_ref, keep_ref, pooled_ref):
    ni = pl.program_id(1)

    @pl.when(ni == 0)
    def _init():
        pooled_ref[...] = jnp.zeros_like(pooled_ref)

    keep = keep_ref[...]
    x = x_ref[...]
    pooled_ref[...] += jnp.sum(x * keep[:, :, None], axis=1)


def _proj_kernel(pooled_ref, keep_ref, hef_ref, w1_ref, b1_ref, w2_ref, b2_ref,
                 out_ref, xl_ref, h_ref, w2s_ref, *, tile_d, tile_m2):
    di = pl.program_id(0)

    @pl.when(di == 0)
    def _scalars():
        pooled = pooled_ref[...]
        den = jnp.maximum(jnp.sum(keep_ref[...], axis=1, keepdims=True), 1.0)
        mean = pooled / den
        hef = hef_ref[...]
        sf = jnp.sum(jnp.where(hef != 0.0, 0.0, mean), axis=1, keepdims=True)
        sf = jnp.where(sf == 0.0, 1.0, sf)
        scale = 10000.0 / (den * sf)
        xl_ref[...] = jnp.log1p(pooled * scale)
        h_ref[...] = jnp.zeros_like(h_ref)

    xl = xl_ref[:, pl.ds(di * tile_d, tile_d)]
    h_ref[...] += jnp.dot(xl, w1_ref[...], preferred_element_type=jnp.float32)
    w2s_ref[pl.ds(di * tile_m2, tile_m2), :] = w2_ref[...]

    @pl.when(di == pl.num_programs(0) - 1)
    def _finalize():
        h = jnp.maximum(h_ref[...] + b1_ref[...], 0.0)
        out_ref[...] = (
            jnp.dot(h, w2s_ref[...], preferred_element_type=jnp.float32)
            + b2_ref[...]).astype(out_ref.dtype)


def kernel(x, x_mask, he_mask, w1, b1, w2, b2):
    B, N, D = x.shape
    M = w1.shape[1]
    f32 = jnp.float32
    x = x.astype(f32)

    keep = (~x_mask).astype(f32)
    hef = he_mask.astype(f32)[None, :]
    b1r = b1.astype(f32)[None, :]
    b2r = b2.astype(f32)[None, :]
    w1 = w1.astype(f32)
    w2 = w2.astype(f32)

    td = min(D, 512)
    tn = min(N, 512)
    nd, nn = D // td, N // tn
    assert nd * td == D and nn * tn == N

    pooled = pl.pallas_call(
        _pool_kernel,
        out_shape=jax.ShapeDtypeStruct((B, D), f32),
        grid_spec=pltpu.PrefetchScalarGridSpec(
            num_scalar_prefetch=0,
            grid=(nd, nn),
            in_specs=[
                pl.BlockSpec((B, tn, td), lambda di, ni: (0, ni, di)),
                pl.BlockSpec((B, tn), lambda di, ni: (0, ni)),
            ],
            out_specs=pl.BlockSpec((B, td), lambda di, ni: (0, di)),
        ),
        compiler_params=pltpu.CompilerParams(
            dimension_semantics=("parallel", "arbitrary"),
            vmem_limit_bytes=48 * 1024 * 1024,
        ),
    )(x, keep)

    td2 = min(D, 256)
    nd2 = D // td2
    tm2 = M // nd2
    assert tm2 * nd2 == M and tm2 % 8 == 0

    out = pl.pallas_call(
        functools.partial(_proj_kernel, tile_d=td2, tile_m2=tm2),
        out_shape=jax.ShapeDtypeStruct((B, M), f32),
        grid_spec=pltpu.PrefetchScalarGridSpec(
            num_scalar_prefetch=0,
            grid=(nd2,),
            in_specs=[
                pl.BlockSpec((B, D), lambda di: (0, 0)),
                pl.BlockSpec((B, N), lambda di: (0, 0)),
                pl.BlockSpec((1, D), lambda di: (0, 0)),
                pl.BlockSpec((td2, M), lambda di: (di, 0)),
                pl.BlockSpec((1, M), lambda di: (0, 0)),
                pl.BlockSpec((tm2, M), lambda di: (di, 0)),
                pl.BlockSpec((1, M), lambda di: (0, 0)),
            ],
            out_specs=pl.BlockSpec((B, M), lambda di: (0, 0)),
            scratch_shapes=[
                pltpu.VMEM((B, D), f32),
                pltpu.VMEM((B, M), f32),
                pltpu.VMEM((M, M), f32),
            ],
        ),
        compiler_params=pltpu.CompilerParams(
            dimension_semantics=("arbitrary",),
            vmem_limit_bytes=48 * 1024 * 1024,
        ),
    )(pooled, keep, hef, w1, b1r, w2, b2r)
    return out


# speedup vs baseline: 1.0299x; 1.0157x over previous
import functools

import jax
import jax.numpy as jnp
from jax.experimental import pallas as pl
from jax.experimental.pallas import tpu as pltpu


def _pool_kernel(x_ref, keep_ref, pooled_ref):
    ni = pl.program_id(1)

    @pl.when(ni == 0)
    def _init():
        pooled_ref[...] = jnp.zeros_like(pooled_ref)

    keep = keep_ref[...]
    x = x_ref[...]
    pooled_ref[...] += jnp.sum(x * keep[:, :, None], axis=1)


def _proj_kernel(pooled_ref, keep_ref, hef_ref, w1_ref, b1_ref, w2_ref, b2_ref,
                 out_ref, xl_ref, h_ref, w2s_ref, *, tile_d, tile_m2):
    di = pl.program_id(0)

    @pl.when(di == 0)
    def _scalars():
        pooled = pooled_ref[...]
        den = jnp.maximum(jnp.sum(keep_ref[...], axis=1, keepdims=True), 1.0)
        mean = pooled / den
        hef = hef_ref[...]
        sf = jnp.sum(jnp.where(hef != 0.0, 0.0, mean), axis=1, keepdims=True)
        sf = jnp.where(sf == 0.0, 1.0, sf)
        scale = 10000.0 / (den * sf)
        xl_ref[...] = jnp.log1p(pooled * scale)
        h_ref[...] = jnp.zeros_like(h_ref)

    xl = xl_ref[:, pl.ds(di * tile_d, tile_d)]
    h_ref[...] += jnp.dot(xl, w1_ref[...], preferred_element_type=jnp.float32)
    w2s_ref[pl.ds(di * tile_m2, tile_m2), :] = w2_ref[...]

    @pl.when(di == pl.num_programs(0) - 1)
    def _finalize():
        h = jnp.maximum(h_ref[...] + b1_ref[...], 0.0)
        out_ref[...] = (
            jnp.dot(h, w2s_ref[...], preferred_element_type=jnp.float32)
            + b2_ref[...]).astype(out_ref.dtype)


def kernel(x, x_mask, he_mask, w1, b1, w2, b2):
    B, N, D = x.shape
    M = w1.shape[1]
    f32 = jnp.float32
    x = x.astype(f32)

    keep = (~x_mask).astype(f32)
    hef = he_mask.astype(f32)[None, :]
    b1r = b1.astype(f32)[None, :]
    b2r = b2.astype(f32)[None, :]
    w1 = w1.astype(f32)
    w2 = w2.astype(f32)

    td = min(D, 512)
    tn = min(N, 512)
    nd, nn = D // td, N // tn
    assert nd * td == D and nn * tn == N

    pooled = pl.pallas_call(
        _pool_kernel,
        out_shape=jax.ShapeDtypeStruct((B, D), f32),
        grid_spec=pltpu.PrefetchScalarGridSpec(
            num_scalar_prefetch=0,
            grid=(nd, nn),
            in_specs=[
                pl.BlockSpec((B, tn, td), lambda di, ni: (0, ni, di)),
                pl.BlockSpec((B, tn), lambda di, ni: (0, ni)),
            ],
            out_specs=pl.BlockSpec((B, td), lambda di, ni: (0, di)),
        ),
        compiler_params=pltpu.CompilerParams(
            dimension_semantics=("parallel", "arbitrary"),
            vmem_limit_bytes=48 * 1024 * 1024,
        ),
    )(x, keep)

    td2 = min(D, 512)
    nd2 = D // td2
    tm2 = M // nd2
    assert tm2 * nd2 == M and tm2 % 8 == 0

    out = pl.pallas_call(
        functools.partial(_proj_kernel, tile_d=td2, tile_m2=tm2),
        out_shape=jax.ShapeDtypeStruct((B, M), f32),
        grid_spec=pltpu.PrefetchScalarGridSpec(
            num_scalar_prefetch=0,
            grid=(nd2,),
            in_specs=[
                pl.BlockSpec((B, D), lambda di: (0, 0)),
                pl.BlockSpec((B, N), lambda di: (0, 0)),
                pl.BlockSpec((1, D), lambda di: (0, 0)),
                pl.BlockSpec((td2, M), lambda di: (di, 0)),
                pl.BlockSpec((1, M), lambda di: (0, 0)),
                pl.BlockSpec((tm2, M), lambda di: (di, 0)),
                pl.BlockSpec((1, M), lambda di: (0, 0)),
            ],
            out_specs=pl.BlockSpec((B, M), lambda di: (0, 0)),
            scratch_shapes=[
                pltpu.VMEM((B, D), f32),
                pltpu.VMEM((B, M), f32),
                pltpu.VMEM((M, M), f32),
            ],
        ),
        compiler_params=pltpu.CompilerParams(
            dimension_semantics=("arbitrary",),
            vmem_limit_bytes=48 * 1024 * 1024,
        ),
    )(pooled, keep, hef, w1, b1r, w2, b2r)
    return out
